# baseline (speedup 1.0000x reference)
import functools

import jax
import jax.numpy as jnp
from jax import lax
from jax.experimental import pallas as pl
from jax.experimental.pallas import tpu as pltpu
from jax.experimental.pallas import tpu_sc as plsc

_B = 64
_NCLS = 13
_PADC = 16
_L = 16
_RPT = 16
_NT = _B // _RPT

_mesh = plsc.VectorSubcoreMesh(
    core_axis_name="c", subcore_axis_name="s", num_cores=1
)


@functools.partial(
    pl.kernel,
    mesh=_mesh,
    out_type=jax.ShapeDtypeStruct((_B, _NCLS), jnp.float32),
    scratch_types=[
        pltpu.VMEM((_B,), jnp.int32),
        pltpu.VMEM((_RPT, _PADC), jnp.float32),
        pltpu.SemaphoreType.DMA,
        pltpu.SemaphoreType.DMA,
    ],
    compiler_params=pltpu.CompilerParams(
        needs_layout_passes=False,
        skip_device_barrier=True,
        disable_semaphore_checks=True,
        disable_bounds_checks=True,
    ),
)
def _scatter_logits(labels_hbm, out_hbm, labels_v, buf_v, sem, lsem):
    w = lax.axis_index("s")

    @pl.when(w < _NT)
    def _():
        lcp = pltpu.make_async_copy(labels_hbm, labels_v, lsem)
        lcp.start()

        neg = jnp.full((_L,), -10.0, jnp.float32)
        for j in range(_RPT):
            buf_v[j, :] = neg
        lcp.wait()

        chunk = labels_v[pl.ds(w * _L, _L)]
        iota = lax.iota(jnp.int32, _L)
        ten = jnp.full((_L,), 10.0, jnp.float32)
        plsc.store_scatter(buf_v, [iota, chunk], ten)

        cps = [
            pltpu.make_async_copy(
                buf_v.at[r % _RPT, pl.ds(0, _NCLS)],
                out_hbm.at[r],
                sem,
            )
            for r in range(_B)
        ]
        for t in range(_NT):
            @pl.when(w == t)
            def _(t=t):
                for j in range(_RPT):
                    cps[t * _RPT + j].start()

        for j in range(_RPT):
            cps[j].wait()


def kernel(x, labels):
    del x
    return _scatter_logits(labels)


# speedup vs baseline: 1.0362x; 1.0362x over previous
import functools

import jax
import jax.numpy as jnp
from jax import lax
from jax.experimental import pallas as pl
from jax.experimental.pallas import tpu as pltpu
from jax.experimental.pallas import tpu_sc as plsc

_B = 64
_NCLS = 13
_L = 16

_mesh = plsc.VectorSubcoreMesh(
    core_axis_name="c", subcore_axis_name="s", num_cores=1
)


@functools.partial(
    pl.kernel,
    mesh=_mesh,
    out_type=jax.ShapeDtypeStruct((_B, _NCLS), jnp.float32),
    scratch_types=[
        pltpu.VMEM((_B,), jnp.int32),
        pltpu.VMEM((_B, _NCLS), jnp.float32),
        pltpu.SemaphoreType.DMA,
    ],
    compiler_params=pltpu.CompilerParams(
        needs_layout_passes=False,
        skip_device_barrier=True,
        disable_semaphore_checks=True,
        disable_bounds_checks=True,
        use_tc_tiling_on_sc=False,
    ),
)
def _scatter_logits(labels_hbm, out_hbm, labels_v, buf_v, lsem):
    w = lax.axis_index("s")

    @pl.when(w == 0)
    def _():
        lcp = pltpu.make_async_copy(labels_hbm, labels_v, lsem)
        lcp.start()

        iota = lax.iota(jnp.int32, _L)
        cmask = iota < _NCLS
        neg = jnp.full((_L,), -10.0, jnp.float32)
        for i in range(_B):
            plsc.store_scatter(
                buf_v, [jnp.full((_L,), i, jnp.int32), iota], neg, mask=cmask
            )
        lcp.wait()

        ten = jnp.full((_L,), 10.0, jnp.float32)
        for k in range(_B // _L):
            chunk = labels_v[pl.ds(k * _L, _L)]
            plsc.store_scatter(buf_v, [iota + k * _L, chunk], ten)

        pltpu.sync_copy(buf_v, out_hbm)


def kernel(x, labels):
    del x
    return _scatter_logits(labels)


# speedup vs baseline: 1.0366x; 1.0004x over previous
import functools

import jax
import jax.numpy as jnp
from jax import lax
from jax.experimental import pallas as pl
from jax.experimental.pallas import tpu as pltpu
from jax.experimental.pallas import tpu_sc as plsc

_B = 64
_NCLS = 13
_L = 16
_RPT = 16
_NT = _B // _RPT

_mesh = plsc.VectorSubcoreMesh(
    core_axis_name="c", subcore_axis_name="s", num_cores=1
)


@functools.partial(
    pl.kernel,
    mesh=_mesh,
    out_type=jax.ShapeDtypeStruct((_B, _NCLS), jnp.float32),
    scratch_types=[
        pltpu.VMEM((_B,), jnp.int32),
        pltpu.VMEM((_RPT, _NCLS), jnp.float32),
        pltpu.SemaphoreType.DMA,
    ],
    compiler_params=pltpu.CompilerParams(
        needs_layout_passes=False,
        skip_device_barrier=True,
        disable_semaphore_checks=True,
        disable_bounds_checks=True,
        use_tc_tiling_on_sc=False,
    ),
)
def _scatter_logits(labels_hbm, out_hbm, labels_v, buf_v, lsem):
    w = lax.axis_index("s")

    @pl.when(w < _NT)
    def _():
        lcp = pltpu.make_async_copy(labels_hbm, labels_v, lsem)
        lcp.start()

        iota = lax.iota(jnp.int32, _L)
        neg = jnp.full((_L,), -10.0, jnp.float32)
        for c in range(_NCLS):
            plsc.store_scatter(
                buf_v, [iota, jnp.full((_L,), c, jnp.int32)], neg
            )
        lcp.wait()

        chunk = labels_v[pl.ds(w * _L, _L)]
        ten = jnp.full((_L,), 10.0, jnp.float32)
        plsc.store_scatter(buf_v, [iota, chunk], ten)

        pltpu.sync_copy(buf_v, out_hbm.at[pl.ds(w * _RPT, _RPT)])


def kernel(x, labels):
    del x
    return _scatter_logits(labels)


# speedup vs baseline: 1.0469x; 1.0099x over previous
import functools

import jax
import jax.numpy as jnp
from jax import lax
from jax.experimental import pallas as pl
from jax.experimental.pallas import tpu as pltpu
from jax.experimental.pallas import tpu_sc as plsc

_B = 64
_NCLS = 13
_L = 16
_RPT = 16
_NT = _B // _RPT

_mesh = plsc.VectorSubcoreMesh(
    core_axis_name="c", subcore_axis_name="s", num_cores=1
)


@functools.partial(
    pl.kernel,
    mesh=_mesh,
    out_type=jax.ShapeDtypeStruct((_B, _NCLS), jnp.float32),
    scratch_types=[
        pltpu.VMEM((_L,), jnp.int32),
        pltpu.VMEM((_RPT, _NCLS), jnp.float32),
        pltpu.SemaphoreType.DMA,
    ],
    compiler_params=pltpu.CompilerParams(
        needs_layout_passes=False,
        skip_device_barrier=True,
        disable_semaphore_checks=True,
        disable_bounds_checks=True,
        use_tc_tiling_on_sc=False,
    ),
)
def _scatter_logits(labels_hbm, out_hbm, labels_v, buf_v, lsem):
    w = lax.axis_index("s")

    @pl.when(w < _NT)
    def _():
        lcp = pltpu.make_async_copy(
            labels_hbm.at[pl.ds(w * _L, _L)], labels_v, lsem
        )
        lcp.start()

        iota = lax.iota(jnp.int32, _L)
        neg = jnp.full((_L,), -10.0, jnp.float32)
        for c in range(_NCLS):
            plsc.store_scatter(
                buf_v, [iota, jnp.full((_L,), c, jnp.int32)], neg
            )
        lcp.wait()

        chunk = labels_v[:]
        ten = jnp.full((_L,), 10.0, jnp.float32)
        plsc.store_scatter(buf_v, [iota, chunk], ten)

        pltpu.sync_copy(buf_v, out_hbm.at[pl.ds(w * _RPT, _RPT)])


def kernel(x, labels):
    del x
    return _scatter_logits(labels)
